# baseline (speedup 1.0000x reference)
import functools

import jax
import jax.numpy as jnp
from jax import lax
from jax.experimental import pallas as pl
from jax.experimental.pallas import tpu as pltpu
from jax.experimental.pallas import tpu_sc as plsc

MAXLEN = 200
EMBED = 32
LANES = 16
IDX_PAD = 208


def kernel(x, token_table, pos_table):
    xn = x[0]

    mesh = plsc.VectorSubcoreMesh(core_axis_name="c", subcore_axis_name="s")
    info = plsc.get_sparse_core_info()
    num_cores = info.num_cores

    @functools.partial(
        pl.kernel,
        mesh=mesh,
        out_type=jax.ShapeDtypeStruct((MAXLEN, EMBED), jnp.float32),
        scratch_types=[
            pltpu.VMEM((IDX_PAD,), jnp.int32),
            pltpu.VMEM((MAXLEN, EMBED), jnp.float32),
            pltpu.VMEM((MAXLEN, EMBED), jnp.float32),
            pltpu.SemaphoreType.DMA,
        ],
    )
    def _embed(idx_hbm, tok_hbm, pos_hbm, out_hbm, idx_v, rows_v, pos_v, sem):
        wid = lax.axis_index("s") * num_cores + lax.axis_index("c")

        @pl.when(wid == 0)
        def _():
            pltpu.sync_copy(idx_hbm, idx_v.at[pl.ds(0, MAXLEN)])
            copies = []
            for b in range(MAXLEN // LANES + 1):
                iv = idx_v[pl.ds(b * LANES, LANES)]
                for l in range(LANES):
                    r = b * LANES + l
                    if r >= MAXLEN:
                        break
                    copies.append(
                        pltpu.async_copy(tok_hbm.at[iv[l]], rows_v.at[r], sem)
                    )
            pltpu.sync_copy(pos_hbm, pos_v)
            for c in copies:
                c.wait()
            for r in range(MAXLEN):
                for c in range(EMBED // LANES):
                    sl = pl.ds(c * LANES, LANES)
                    rows_v[r, sl] = rows_v[r, sl] + pos_v[r, sl]
            pltpu.sync_copy(rows_v, out_hbm)

    return _embed(xn, token_table, pos_table)


# speedup vs baseline: 1.0757x; 1.0757x over previous
import jax
import jax.numpy as jnp
from jax.experimental import pallas as pl
from jax.experimental.pallas import tpu as pltpu

MAXLEN = 200
EMBED = 32


def _body(xn_ref, tok_ref, pos_ref, out_ref, rows_ref, sem):
    copies = []
    for r in range(MAXLEN):
        c = pltpu.make_async_copy(
            tok_ref.at[xn_ref[r]], rows_ref.at[r], sem
        )
        c.start()
        copies.append(c)
    for c in copies:
        c.wait()
    out_ref[...] = rows_ref[...] + pos_ref[...]


def kernel(x, token_table, pos_table):
    xn = x[0]
    return pl.pallas_call(
        _body,
        out_shape=jax.ShapeDtypeStruct((MAXLEN, EMBED), jnp.float32),
        in_specs=[
            pl.BlockSpec(memory_space=pltpu.SMEM),
            pl.BlockSpec(memory_space=pl.ANY),
            pl.BlockSpec(memory_space=pltpu.VMEM),
        ],
        out_specs=pl.BlockSpec(memory_space=pltpu.VMEM),
        scratch_shapes=[
            pltpu.VMEM((MAXLEN, EMBED), jnp.float32),
            pltpu.SemaphoreType.DMA,
        ],
    )(xn, token_table, pos_table)
